# Initial kernel scaffold; baseline (speedup 1.0000x reference)
import jax
import jax.numpy as jnp
from jax.experimental import pallas as pl


def kernel(z, embedding):
    raise NotImplementedError("write your pallas kernel here")


# speedup vs baseline: 2.0745x; 2.0745x over previous
import functools

import jax
import jax.numpy as jnp
from jax.experimental import pallas as pl

_CODEBOOK = 1024
_D = 256
_ROWS_PER_BLOCK = 2048


def _vq_block(z_ref, emb_ref, zq_ref, idx_ref, acc_ref):
    i = pl.program_id(0)
    zb = z_ref[...]
    emb = emb_ref[...]
    e_sq = jnp.sum(emb * emb, axis=1)[None, :]
    z_sq = jnp.sum(zb * zb, axis=1, keepdims=True)
    scores = jax.lax.dot_general(
        zb, emb, (((1,), (1,)), ((), ())),
        preferred_element_type=jnp.float32)
    d = z_sq + e_sq - 2.0 * scores
    min_d = jnp.min(d, axis=1, keepdims=True)
    col = jax.lax.broadcasted_iota(jnp.int32, d.shape, 1)
    idx = jnp.min(jnp.where(d == min_d, col, jnp.int32(_CODEBOOK)), axis=1)
    oh = (col == idx[:, None]).astype(jnp.float32)
    zq = jnp.dot(oh, emb, preferred_element_type=jnp.float32)
    zq_ref[...] = zq
    idx_ref[0, 0, :] = idx
    diff = zq - zb
    part = jnp.sum(diff * diff).reshape(1, 1)

    @pl.when(i == 0)
    def _init():
        acc_ref[...] = part

    @pl.when(i != 0)
    def _accum():
        acc_ref[...] += part


@functools.partial(jax.jit, static_argnames=())
def kernel(z, embedding):
    z = z.astype(jnp.float32)
    B, T, D = z.shape
    N = B * T
    R = _ROWS_PER_BLOCK
    nb = N // R
    z_flat = z.reshape(N, D)

    zq, idx3, acc = pl.pallas_call(
        _vq_block,
        grid=(nb,),
        in_specs=[
            pl.BlockSpec((R, D), lambda i: (i, 0)),
            pl.BlockSpec((_CODEBOOK, D), lambda i: (0, 0)),
        ],
        out_specs=[
            pl.BlockSpec((R, D), lambda i: (i, 0)),
            pl.BlockSpec((1, 1, R), lambda i: (i, 0, 0)),
            pl.BlockSpec((1, 1), lambda i: (0, 0)),
        ],
        out_shape=[
            jax.ShapeDtypeStruct((N, D), jnp.float32),
            jax.ShapeDtypeStruct((nb, 1, R), jnp.int32),
            jax.ShapeDtypeStruct((1, 1), jnp.float32),
        ],
    )(z_flat, embedding)

    z_quantized = zq.reshape(B, T, D)
    indices = idx3.reshape(B, T)
    m = acc[0, 0] / jnp.float32(N * D)
    commitment_loss = jnp.float32(0.25) * m
    codebook_loss = m
    loss = commitment_loss + codebook_loss
    return (z_quantized, loss, commitment_loss, codebook_loss, indices)
